# baseline (speedup 1.0000x reference)
import functools

import jax
import jax.numpy as jnp
from jax import lax
from jax.experimental import pallas as pl
from jax.experimental.pallas import tpu as pltpu
from jax.experimental.pallas import tpu_sc as plsc

NUM_EMBEDDINGS = 1000000
D = 32
BATCH = 4096
HIST_LEN = 200
B = BATCH * HIST_LEN

NC = 2
NS = 16
NW = NC * NS
G = 128
GROUPS = B // (NW * G)
NR = 4
NT = 2


def _make_gather_t():
    mesh = plsc.VectorSubcoreMesh(core_axis_name="c", subcore_axis_name="s")

    @functools.partial(
        pl.kernel,
        mesh=mesh,
        out_type=jax.ShapeDtypeStruct((B * D,), jnp.float32),
        compiler_params=pltpu.CompilerParams(
            use_tc_tiling_on_sc=False, needs_layout_passes=False
        ),
        scratch_types=[
            pltpu.VMEM((GROUPS, G), jnp.int32),
            pltpu.VMEM((NR, G, D), jnp.float32),
            pltpu.VMEM((NT, D * G), jnp.float32),
            pltpu.SemaphoreType.DMA,
            pltpu.SemaphoreType.DMA,
        ],
    )
    def gather_t(idx_hbm, table_hbm, out_hbm, idx_v, rows_v, trans_v, gsem, osem):
        wid = lax.axis_index("s") * NC + lax.axis_index("c")
        base = wid * GROUPS
        pltpu.sync_copy(idx_hbm.at[pl.ds(base, GROUPS)], idx_v)

        iota = lax.iota(jnp.int32, 16)
        row_base = [(iota + c * 16) * D for c in range(8)]

        pltpu.async_copy(table_hbm.at[idx_v.at[0]], rows_v.at[0], gsem)

        def step(g, carry):
            rb = lax.rem(g, NR)
            tb = lax.rem(g, NT)

            @pl.when(g >= NT)
            def _():
                pltpu.make_async_copy(
                    trans_v.at[tb],
                    out_hbm.at[pl.ds((base + g - NT) * G * D, G * D)],
                    osem,
                ).wait()

            @pl.when(g + 1 < GROUPS)
            def _():
                pltpu.async_copy(
                    table_hbm.at[idx_v.at[g + 1]],
                    rows_v.at[lax.rem(g + 1, NR)],
                    gsem,
                )

            pltpu.make_async_copy(
                table_hbm.at[idx_v.at[g]], rows_v.at[rb], gsem
            ).wait()

            rv = rows_v.at[rb]
            tv = trans_v.at[tb]
            for d in range(D):
                for c in range(8):
                    x = plsc.load_gather(rv, [iota + c * 16, jnp.full((16,), d, jnp.int32)])
                    tv[pl.ds(d * G + c * 16, 16)] = x

            pltpu.async_copy(
                tv, out_hbm.at[pl.ds((base + g) * G * D, G * D)], osem
            )
            return carry

        lax.fori_loop(0, GROUPS, step, 0)

        for g in (GROUPS - NT, GROUPS - 1):
            pltpu.make_async_copy(
                trans_v.at[g % NT],
                out_hbm.at[pl.ds((base + g) * G * D, G * D)],
                osem,
            ).wait()

    return gather_t


def _make_retile():
    mesh = plsc.VectorSubcoreMesh(core_axis_name="c", subcore_axis_name="s")
    NB = 4
    BT = BATCH // G

    @functools.partial(
        pl.kernel,
        mesh=mesh,
        out_type=jax.ShapeDtypeStruct((HIST_LEN, D, BATCH), jnp.float32),
        compiler_params=pltpu.CompilerParams(use_tc_tiling_on_sc=True),
        scratch_types=[
            pltpu.VMEM((NB, D, G), jnp.float32),
            pltpu.SemaphoreType.DMA,
            pltpu.SemaphoreType.DMA,
        ],
    )
    def retile(in_hbm, out_hbm, buf, isem, osem):
        wid = lax.axis_index("s") * NC + lax.axis_index("c")
        base = wid * GROUPS

        def out_slice(j):
            t = lax.div(j, BT)
            bt = lax.rem(j, BT)
            return out_hbm.at[t, :, pl.ds(bt * G, G)]

        pltpu.async_copy(in_hbm.at[base], buf.at[0], isem)

        def step(g, carry):
            b = lax.rem(g, NB)

            @pl.when(g >= NB - 1)
            def _():
                pltpu.make_async_copy(
                    buf.at[lax.rem(g + 1, NB)], out_slice(base + g + 1 - NB), osem
                ).wait()

            @pl.when(g + 1 < GROUPS)
            def _():
                pltpu.async_copy(
                    in_hbm.at[base + g + 1], buf.at[lax.rem(g + 1, NB)], isem
                )

            pltpu.make_async_copy(in_hbm.at[base + g], buf.at[b], isem).wait()
            pltpu.async_copy(buf.at[b], out_slice(base + g), osem)
            return carry

        lax.fori_loop(0, GROUPS, step, 0)

        for g in range(GROUPS - NB + 1, GROUPS):
            pltpu.make_async_copy(
                buf.at[g % NB], out_slice(base + g), osem
            ).wait()

    return retile


_gather_t = _make_gather_t()
_retile = _make_retile()


def kernel(token_ids, W):
    idx = jnp.transpose(token_ids).reshape(NW * GROUPS, G).astype(jnp.int32)
    flat = _gather_t(idx, W)
    blocks = flat.reshape(NW * GROUPS, D, G)
    out3 = _retile(blocks)
    return jnp.transpose(out3, (2, 0, 1))


# speedup vs baseline: 1.1141x; 1.1141x over previous
import functools

import jax
import jax.numpy as jnp
from jax import lax
from jax.experimental import pallas as pl
from jax.experimental.pallas import tpu as pltpu
from jax.experimental.pallas import tpu_sc as plsc

NUM_EMBEDDINGS = 1000000
D = 32
BATCH = 4096
HIST_LEN = 200
B = BATCH * HIST_LEN

NC = 2
NS = 16
NW = NC * NS
G = 128
GROUPS = B // (NW * G)
NR = 4
NT = 2


def _make_gather_t():
    mesh = plsc.VectorSubcoreMesh(core_axis_name="c", subcore_axis_name="s")

    @functools.partial(
        pl.kernel,
        mesh=mesh,
        out_type=jax.ShapeDtypeStruct((B * D,), jnp.float32),
        compiler_params=pltpu.CompilerParams(
            use_tc_tiling_on_sc=False, needs_layout_passes=False
        ),
        scratch_types=[
            pltpu.VMEM((GROUPS, G), jnp.int32),
            pltpu.VMEM((NR, G, D), jnp.float32),
            pltpu.VMEM((NT, D * G), jnp.float32),
            pltpu.SemaphoreType.DMA,
            pltpu.SemaphoreType.DMA,
        ],
    )
    def gather_t(idx_hbm, table_hbm, out_hbm, idx_v, rows_v, trans_v, gsem, osem):
        wid = lax.axis_index("s") * NC + lax.axis_index("c")
        base = wid * GROUPS
        pltpu.sync_copy(idx_hbm.at[pl.ds(base, GROUPS)], idx_v)

        iota = lax.iota(jnp.int32, 16)
        row_base = [(iota + c * 16) * D for c in range(8)]

        pltpu.async_copy(table_hbm.at[idx_v.at[0]], rows_v.at[0], gsem)

        def step(g, carry):
            rb = lax.rem(g, NR)
            tb = lax.rem(g, NT)

            @pl.when(g >= NT)
            def _():
                pltpu.make_async_copy(
                    trans_v.at[tb],
                    out_hbm.at[pl.ds((base + g - NT) * G * D, G * D)],
                    osem,
                ).wait()

            @pl.when(g + 1 < GROUPS)
            def _():
                pltpu.async_copy(
                    table_hbm.at[idx_v.at[g + 1]],
                    rows_v.at[lax.rem(g + 1, NR)],
                    gsem,
                )

            pltpu.make_async_copy(
                table_hbm.at[idx_v.at[g]], rows_v.at[rb], gsem
            ).wait()

            rv = rows_v.at[rb]
            tv = trans_v.at[tb]

            def loads(d0):
                return [
                    (d, c, plsc.load_gather(
                        rv, [iota + c * 16, jnp.full((16,), d, jnp.int32)]))
                    for d in (d0, d0 + 1)
                    for c in range(8)
                ]

            prev = loads(0)
            for d0 in range(2, D, 2):
                cur = loads(d0)
                for d, c, x in prev:
                    tv[pl.ds(d * G + c * 16, 16)] = x
                prev = cur
            for d, c, x in prev:
                tv[pl.ds(d * G + c * 16, 16)] = x

            pltpu.async_copy(
                tv, out_hbm.at[pl.ds((base + g) * G * D, G * D)], osem
            )
            return carry

        lax.fori_loop(0, GROUPS, step, 0)

        for g in (GROUPS - NT, GROUPS - 1):
            pltpu.make_async_copy(
                trans_v.at[g % NT],
                out_hbm.at[pl.ds((base + g) * G * D, G * D)],
                osem,
            ).wait()

    return gather_t


def _make_retile():
    mesh = plsc.VectorSubcoreMesh(core_axis_name="c", subcore_axis_name="s")
    NB = 4
    BT = BATCH // G

    @functools.partial(
        pl.kernel,
        mesh=mesh,
        out_type=jax.ShapeDtypeStruct((HIST_LEN, D, BATCH), jnp.float32),
        compiler_params=pltpu.CompilerParams(use_tc_tiling_on_sc=True),
        scratch_types=[
            pltpu.VMEM((NB, D, G), jnp.float32),
            pltpu.SemaphoreType.DMA,
            pltpu.SemaphoreType.DMA,
        ],
    )
    def retile(in_hbm, out_hbm, buf, isem, osem):
        wid = lax.axis_index("s") * NC + lax.axis_index("c")
        base = wid * GROUPS

        def out_slice(j):
            t = lax.div(j, BT)
            bt = lax.rem(j, BT)
            return out_hbm.at[t, :, pl.ds(bt * G, G)]

        pltpu.async_copy(in_hbm.at[base], buf.at[0], isem)

        def step(g, carry):
            b = lax.rem(g, NB)

            @pl.when(g >= NB - 1)
            def _():
                pltpu.make_async_copy(
                    buf.at[lax.rem(g + 1, NB)], out_slice(base + g + 1 - NB), osem
                ).wait()

            @pl.when(g + 1 < GROUPS)
            def _():
                pltpu.async_copy(
                    in_hbm.at[base + g + 1], buf.at[lax.rem(g + 1, NB)], isem
                )

            pltpu.make_async_copy(in_hbm.at[base + g], buf.at[b], isem).wait()
            pltpu.async_copy(buf.at[b], out_slice(base + g), osem)
            return carry

        lax.fori_loop(0, GROUPS, step, 0)

        for g in range(GROUPS - NB + 1, GROUPS):
            pltpu.make_async_copy(
                buf.at[g % NB], out_slice(base + g), osem
            ).wait()

    return retile


_gather_t = _make_gather_t()
_retile = _make_retile()


def kernel(token_ids, W):
    idx = jnp.transpose(token_ids).reshape(NW * GROUPS, G).astype(jnp.int32)
    flat = _gather_t(idx, W)
    blocks = flat.reshape(NW * GROUPS, D, G)
    out3 = _retile(blocks)
    return jnp.transpose(out3, (2, 0, 1))


# speedup vs baseline: 1.1507x; 1.0328x over previous
import functools

import jax
import jax.numpy as jnp
from jax import lax
from jax.experimental import pallas as pl
from jax.experimental.pallas import tpu as pltpu
from jax.experimental.pallas import tpu_sc as plsc

NUM_EMBEDDINGS = 1000000
D = 32
BATCH = 4096
HIST_LEN = 200
B = BATCH * HIST_LEN

NC = 2
NS = 16
NW = NC * NS
G = 128
GROUPS = B // (NW * G)
NR = 8
NP = 6
NT = 2


def _make_gather_t():
    mesh = plsc.VectorSubcoreMesh(core_axis_name="c", subcore_axis_name="s")

    @functools.partial(
        pl.kernel,
        mesh=mesh,
        out_type=jax.ShapeDtypeStruct((B * D,), jnp.float32),
        compiler_params=pltpu.CompilerParams(
            use_tc_tiling_on_sc=False, needs_layout_passes=False
        ),
        scratch_types=[
            pltpu.VMEM((GROUPS, G), jnp.int32),
            pltpu.VMEM((NR, G, D), jnp.float32),
            pltpu.VMEM((NT, D * G), jnp.float32),
            pltpu.SemaphoreType.DMA,
            pltpu.SemaphoreType.DMA,
        ],
    )
    def gather_t(idx_hbm, table_hbm, out_hbm, idx_v, rows_v, trans_v, gsem, osem):
        wid = lax.axis_index("s") * NC + lax.axis_index("c")
        base = wid * GROUPS
        pltpu.sync_copy(idx_hbm.at[pl.ds(base, GROUPS)], idx_v)

        iota = lax.iota(jnp.int32, 16)
        row_base = [(iota + c * 16) * D for c in range(8)]

        for p in range(NP):
            pltpu.async_copy(table_hbm.at[idx_v.at[p]], rows_v.at[p], gsem)

        def step(g, carry):
            rb = lax.rem(g, NR)
            tb = lax.rem(g, NT)

            @pl.when(g >= NT)
            def _():
                pltpu.make_async_copy(
                    trans_v.at[tb],
                    out_hbm.at[pl.ds((base + g - NT) * G * D, G * D)],
                    osem,
                ).wait()

            @pl.when(g + NP < GROUPS)
            def _():
                pltpu.async_copy(
                    table_hbm.at[idx_v.at[g + NP]],
                    rows_v.at[lax.rem(g + NP, NR)],
                    gsem,
                )

            pltpu.make_async_copy(
                table_hbm.at[idx_v.at[g]], rows_v.at[rb], gsem
            ).wait()

            rv = rows_v.at[rb]
            tv = trans_v.at[tb]

            def loads(d0):
                return [
                    (d, c, plsc.load_gather(
                        rv, [iota + c * 16, jnp.full((16,), d, jnp.int32)]))
                    for d in (d0, d0 + 1)
                    for c in range(8)
                ]

            prev = loads(0)
            for d0 in range(2, D, 2):
                cur = loads(d0)
                for d, c, x in prev:
                    tv[pl.ds(d * G + c * 16, 16)] = x
                prev = cur
            for d, c, x in prev:
                tv[pl.ds(d * G + c * 16, 16)] = x

            pltpu.async_copy(
                tv, out_hbm.at[pl.ds((base + g) * G * D, G * D)], osem
            )
            return carry

        lax.fori_loop(0, GROUPS, step, 0)

        for g in (GROUPS - NT, GROUPS - 1):
            pltpu.make_async_copy(
                trans_v.at[g % NT],
                out_hbm.at[pl.ds((base + g) * G * D, G * D)],
                osem,
            ).wait()

    return gather_t


def _make_retile():
    mesh = plsc.VectorSubcoreMesh(core_axis_name="c", subcore_axis_name="s")
    NB = 8
    PB = 6
    BT = BATCH // G

    @functools.partial(
        pl.kernel,
        mesh=mesh,
        out_type=jax.ShapeDtypeStruct((HIST_LEN, D, BATCH), jnp.float32),
        compiler_params=pltpu.CompilerParams(use_tc_tiling_on_sc=True),
        scratch_types=[
            pltpu.VMEM((NB, D, G), jnp.float32),
            pltpu.SemaphoreType.DMA,
            pltpu.SemaphoreType.DMA,
        ],
    )
    def retile(in_hbm, out_hbm, buf, isem, osem):
        wid = lax.axis_index("s") * NC + lax.axis_index("c")
        base = wid * GROUPS

        def out_slice(j):
            t = lax.div(j, BT)
            bt = lax.rem(j, BT)
            return out_hbm.at[t, :, pl.ds(bt * G, G)]

        for p in range(PB):
            pltpu.async_copy(in_hbm.at[base + p], buf.at[p], isem)

        def step(g, carry):
            b = lax.rem(g, NB)

            @pl.when(g + PB >= NB)
            def _():
                pltpu.make_async_copy(
                    buf.at[lax.rem(g + PB, NB)], out_slice(base + g + PB - NB), osem
                ).wait()

            @pl.when(g + PB < GROUPS)
            def _():
                pltpu.async_copy(
                    in_hbm.at[base + g + PB], buf.at[lax.rem(g + PB, NB)], isem
                )

            pltpu.make_async_copy(in_hbm.at[base + g], buf.at[b], isem).wait()
            pltpu.async_copy(buf.at[b], out_slice(base + g), osem)
            return carry

        lax.fori_loop(0, GROUPS, step, 0)

        for g in range(GROUPS - NB + PB, GROUPS):
            pltpu.make_async_copy(
                buf.at[g % NB], out_slice(base + g), osem
            ).wait()

    return retile


_gather_t = _make_gather_t()
_retile = _make_retile()


def kernel(token_ids, W):
    idx = jnp.transpose(token_ids).reshape(NW * GROUPS, G).astype(jnp.int32)
    flat = _gather_t(idx, W)
    blocks = flat.reshape(NW * GROUPS, D, G)
    out3 = _retile(blocks)
    return jnp.transpose(out3, (2, 0, 1))


# speedup vs baseline: 1.8372x; 1.5965x over previous
import functools

import jax
import jax.numpy as jnp
from jax import lax
from jax.experimental import pallas as pl
from jax.experimental.pallas import tpu as pltpu
from jax.experimental.pallas import tpu_sc as plsc

NUM_EMBEDDINGS = 1000000
D = 32
BATCH = 4096
HIST_LEN = 200
B = BATCH * HIST_LEN

NC = 2
NS = 16
NW = NC * NS
G = 128
GROUPS = B // (NW * G)
NR = 8
NP = 6
NT = 2


def _make_gather_t():
    mesh = plsc.VectorSubcoreMesh(core_axis_name="c", subcore_axis_name="s")

    @functools.partial(
        pl.kernel,
        mesh=mesh,
        out_type=jax.ShapeDtypeStruct((B * D,), jnp.float32),
        compiler_params=pltpu.CompilerParams(
            use_tc_tiling_on_sc=False, needs_layout_passes=False
        ),
        scratch_types=[
            pltpu.VMEM((GROUPS, G), jnp.int32),
            pltpu.VMEM((NR, G, D), jnp.float32),
            pltpu.VMEM((NT, D * G), jnp.float32),
            pltpu.SemaphoreType.DMA,
            pltpu.SemaphoreType.DMA,
        ],
    )
    def gather_t(idx_hbm, table_hbm, out_hbm, idx_v, rows_v, trans_v, gsem, osem):
        wid = lax.axis_index("s") * NC + lax.axis_index("c")
        base = wid * GROUPS
        pltpu.sync_copy(idx_hbm.at[pl.ds(base, GROUPS)], idx_v)

        iota = lax.iota(jnp.int32, 16)

        for p in range(NP):
            pltpu.async_copy(table_hbm.at[idx_v.at[p]], rows_v.at[p], gsem)

        def step(g, carry):
            rb = lax.rem(g, NR)
            tb = lax.rem(g, NT)

            @pl.when(g >= NT)
            def _():
                pltpu.make_async_copy(
                    trans_v.at[tb],
                    out_hbm.at[pl.ds((base + g - NT) * G * D, G * D)],
                    osem,
                ).wait()

            @pl.when(g + NP < GROUPS)
            def _():
                pltpu.async_copy(
                    table_hbm.at[idx_v.at[g + NP]],
                    rows_v.at[lax.rem(g + NP, NR)],
                    gsem,
                )

            pltpu.make_async_copy(
                table_hbm.at[idx_v.at[g]], rows_v.at[rb], gsem
            ).wait()

            rv = rows_v.at[rb]
            tv = trans_v.at[tb]
            iz = lax.shift_right_arithmetic(idx_v[g, pl.ds(0, 16)], 31)
            iota_p = iota + iz
            subtiles = [(c, h) for c in range(8) for h in range(2)]

            def loads(c, h):
                out = []
                row = iota_p + c * 16
                for d in range(16):
                    cb = lax.bitwise_and(iota_p + d, jnp.int32(15))
                    x = plsc.load_gather(rv, [row, cb + h * 16])
                    out.append((cb, x))
                return out

            def stores(c, h, xs):
                off = iota_p + (h * 16 * G + c * 16)
                for cb, x in xs:
                    plsc.store_scatter(tv, [lax.shift_left(cb, 7) + off], x)

            prev_ch, prev_xs = subtiles[0], loads(*subtiles[0])
            for ch in subtiles[1:]:
                cur = loads(*ch)
                stores(*prev_ch, prev_xs)
                prev_ch, prev_xs = ch, cur
            stores(*prev_ch, prev_xs)

            pltpu.async_copy(
                tv, out_hbm.at[pl.ds((base + g) * G * D, G * D)], osem
            )
            return carry

        lax.fori_loop(0, GROUPS, step, 0)

        for g in (GROUPS - NT, GROUPS - 1):
            pltpu.make_async_copy(
                trans_v.at[g % NT],
                out_hbm.at[pl.ds((base + g) * G * D, G * D)],
                osem,
            ).wait()

    return gather_t


def _make_retile():
    mesh = plsc.VectorSubcoreMesh(core_axis_name="c", subcore_axis_name="s")
    NB = 8
    PB = 6
    BT = BATCH // G

    @functools.partial(
        pl.kernel,
        mesh=mesh,
        out_type=jax.ShapeDtypeStruct((HIST_LEN, D, BATCH), jnp.float32),
        compiler_params=pltpu.CompilerParams(use_tc_tiling_on_sc=True),
        scratch_types=[
            pltpu.VMEM((NB, D, G), jnp.float32),
            pltpu.SemaphoreType.DMA,
            pltpu.SemaphoreType.DMA,
        ],
    )
    def retile(in_hbm, out_hbm, buf, isem, osem):
        wid = lax.axis_index("s") * NC + lax.axis_index("c")
        base = wid * GROUPS

        def out_slice(j):
            t = lax.div(j, BT)
            bt = lax.rem(j, BT)
            return out_hbm.at[t, :, pl.ds(bt * G, G)]

        for p in range(PB):
            pltpu.async_copy(in_hbm.at[base + p], buf.at[p], isem)

        def step(g, carry):
            b = lax.rem(g, NB)

            @pl.when(g + PB >= NB)
            def _():
                pltpu.make_async_copy(
                    buf.at[lax.rem(g + PB, NB)], out_slice(base + g + PB - NB), osem
                ).wait()

            @pl.when(g + PB < GROUPS)
            def _():
                pltpu.async_copy(
                    in_hbm.at[base + g + PB], buf.at[lax.rem(g + PB, NB)], isem
                )

            pltpu.make_async_copy(in_hbm.at[base + g], buf.at[b], isem).wait()
            pltpu.async_copy(buf.at[b], out_slice(base + g), osem)
            return carry

        lax.fori_loop(0, GROUPS, step, 0)

        for g in range(GROUPS - NB + PB, GROUPS):
            pltpu.make_async_copy(
                buf.at[g % NB], out_slice(base + g), osem
            ).wait()

    return retile


_gather_t = _make_gather_t()
_retile = _make_retile()


def kernel(token_ids, W):
    idx = jnp.transpose(token_ids).reshape(NW * GROUPS, G).astype(jnp.int32)
    flat = _gather_t(idx, W)
    blocks = flat.reshape(NW * GROUPS, D, G)
    out3 = _retile(blocks)
    return jnp.transpose(out3, (2, 0, 1))


# speedup vs baseline: 2.9078x; 1.5827x over previous
import functools

import jax
import jax.numpy as jnp
from jax import lax
from jax.experimental import pallas as pl
from jax.experimental.pallas import tpu as pltpu
from jax.experimental.pallas import tpu_sc as plsc

NUM_EMBEDDINGS = 1000000
D = 32
BATCH = 4096
HIST_LEN = 200
B = BATCH * HIST_LEN

NC = 2
NS = 16
NW = NC * NS
G = 128
GROUPS = B // (NW * G)
NR = 8
NP = 6
NT = 2


def _make_gather_t():
    mesh = plsc.VectorSubcoreMesh(core_axis_name="c", subcore_axis_name="s")

    @functools.partial(
        pl.kernel,
        mesh=mesh,
        out_type=jax.ShapeDtypeStruct((B * D,), jnp.float32),
        compiler_params=pltpu.CompilerParams(
            use_tc_tiling_on_sc=False, needs_layout_passes=False
        ),
        scratch_types=[
            pltpu.VMEM((GROUPS, G), jnp.int32),
            pltpu.VMEM((NR, G, D), jnp.float32),
            pltpu.VMEM((NT, D * G), jnp.float32),
            pltpu.SemaphoreType.DMA,
            pltpu.SemaphoreType.DMA,
        ],
    )
    def gather_t(idx_hbm, table_hbm, out_hbm, idx_v, rows_v, trans_v, gsem, osem):
        wid = lax.axis_index("s") * NC + lax.axis_index("c")
        base = wid * GROUPS
        pltpu.sync_copy(idx_hbm.at[pl.ds(base, GROUPS)], idx_v)

        iota = lax.iota(jnp.int32, 16)

        for p in range(NP):
            pltpu.async_copy(table_hbm.at[idx_v.at[p]], rows_v.at[p], gsem)

        def step(g, carry):
            rb = lax.rem(g, NR)
            tb = lax.rem(g, NT)

            @pl.when(g >= NT)
            def _():
                pltpu.make_async_copy(
                    trans_v.at[tb],
                    out_hbm.at[pl.ds((base + g - NT) * G * D, G * D)],
                    osem,
                ).wait()

            @pl.when(g + NP < GROUPS)
            def _():
                pltpu.async_copy(
                    table_hbm.at[idx_v.at[g + NP]],
                    rows_v.at[lax.rem(g + NP, NR)],
                    gsem,
                )

            pltpu.make_async_copy(
                table_hbm.at[idx_v.at[g]], rows_v.at[rb], gsem
            ).wait()

            rv = rows_v.at[rb]
            tv = trans_v.at[tb]
            iz = lax.shift_right_arithmetic(idx_v[g, pl.ds(0, 16)], 31)
            iota_p = iota + iz
            subtiles = [(c, h) for c in range(8) for h in range(2)]

            def loads(c, h):
                out = []
                row = iota_p + c * 16
                for d in range(16):
                    cb = lax.bitwise_and(iota_p + d, jnp.int32(15))
                    x = plsc.load_gather(rv, [row, cb + h * 16])
                    out.append((cb, x))
                return out

            def stores(c, h, xs):
                off = iota_p + (h * 16 * G + c * 16)
                for cb, x in xs:
                    plsc.store_scatter(tv, [lax.shift_left(cb, 7) + off], x)

            prev_ch, prev_xs = subtiles[0], loads(*subtiles[0])
            for ch in subtiles[1:]:
                cur = loads(*ch)
                stores(*prev_ch, prev_xs)
                prev_ch, prev_xs = ch, cur
            stores(*prev_ch, prev_xs)

            pltpu.async_copy(
                tv, out_hbm.at[pl.ds((base + g) * G * D, G * D)], osem
            )
            return carry

        lax.fori_loop(0, GROUPS, step, 0)

        for g in (GROUPS - NT, GROUPS - 1):
            pltpu.make_async_copy(
                trans_v.at[g % NT],
                out_hbm.at[pl.ds((base + g) * G * D, G * D)],
                osem,
            ).wait()

    return gather_t


def _make_format():
    mesh = plsc.VectorSubcoreMesh(core_axis_name="c", subcore_axis_name="s")
    CHUNKS = NUM_EMBEDDINGS // G
    TAIL = NUM_EMBEDDINGS - CHUNKS * G
    NK = 8
    KP = 4
    NO = 2

    @functools.partial(
        pl.kernel,
        mesh=mesh,
        out_type=jax.ShapeDtypeStruct((NUM_EMBEDDINGS * D,), jnp.float32),
        compiler_params=pltpu.CompilerParams(
            use_tc_tiling_on_sc=True, needs_layout_passes=False
        ),
        scratch_types=[
            pltpu.VMEM((NK * D, G), jnp.float32),
            pltpu.VMEM((NO * D * G,), jnp.float32),
            pltpu.VMEM((NK * 16,), jnp.int32),
            pltpu.VMEM((TAIL * D,), jnp.float32),
            pltpu.SemaphoreType.DMA,
            pltpu.SemaphoreType.DMA,
        ],
    )
    def format_w(wt_hbm, tail_hbm, z_hbm, out_hbm, inb, outb, zbuf, tailv, isem, osem):
        wid = lax.axis_index("s") * NC + lax.axis_index("c")
        pltpu.sync_copy(z_hbm, zbuf)

        @pl.when(wid == NW - 1)
        def _():
            pltpu.sync_copy(tail_hbm, tailv)
            pltpu.sync_copy(tailv, out_hbm.at[pl.ds(CHUNKS * G * D, TAIL * D)])

        nw = jnp.where(wid < CHUNKS - (CHUNKS // NW) * NW, CHUNKS // NW + 1,
                       CHUNKS // NW).astype(jnp.int32)
        iota = lax.iota(jnp.int32, 16)

        def src(i):
            return wt_hbm.at[:, pl.ds((i * NW + wid) * G, G)]

        for p2 in range(KP):
            pltpu.async_copy(src(p2), inb.at[pl.ds(p2 * D, D)], isem)

        def step(i, carry):
            rb = lax.rem(i, NK)
            ob = lax.rem(i, NO)

            @pl.when(i >= NO)
            def _():
                pltpu.make_async_copy(
                    outb.at[pl.ds(ob * D * G, D * G)],
                    out_hbm.at[pl.ds(((i - NO) * NW + wid) * G * D, G * D)],
                    osem,
                ).wait()

            @pl.when(i + KP < nw)
            def _():
                pltpu.async_copy(src(i + KP), inb.at[pl.ds(lax.rem(i + KP, NK) * D, D)], isem)

            pltpu.make_async_copy(src(i), inb.at[pl.ds(rb * D, D)], isem).wait()

            rv = inb.at[pl.ds(rb * D, D)]
            tv = outb.at[pl.ds(ob * D * G, D * G)]
            iz = lax.shift_right_arithmetic(
                zbuf[pl.ds(lax.rem(i, NK) * 16, 16)], 31)
            iota_p = iota + iz
            subtiles = [(rg, cg) for rg in range(2) for cg in range(8)]

            def loads(rg, cg):
                out = []
                row = iota_p + rg * 16
                for k in range(16):
                    cb = lax.bitwise_and(iota_p + k, jnp.int32(15))
                    x = plsc.load_gather(rv, [row, cb + cg * 16])
                    out.append((cb, x))
                return out

            def stores(rg, cg, xs):
                off = iota_p + rg * 16
                for cb, x in xs:
                    sidx = lax.shift_left(cb + cg * 16, 5) + off
                    plsc.store_scatter(tv, [sidx], x)

            prev_ch, prev_xs = subtiles[0], loads(*subtiles[0])
            for ch in subtiles[1:]:
                cur = loads(*ch)
                stores(*prev_ch, prev_xs)
                prev_ch, prev_xs = ch, cur
            stores(*prev_ch, prev_xs)

            pltpu.async_copy(
                tv, out_hbm.at[pl.ds((i * NW + wid) * G * D, G * D)], osem
            )
            return carry

        lax.fori_loop(0, nw, step, jnp.int32(0))

        for back in (NO, 1):
            pltpu.make_async_copy(
                outb.at[pl.ds(lax.rem(nw - back, NO) * D * G, D * G)],
                out_hbm.at[pl.ds(((nw - back) * NW + wid) * G * D, G * D)],
                osem,
            ).wait()

    return format_w


def _make_retile():
    mesh = plsc.VectorSubcoreMesh(core_axis_name="c", subcore_axis_name="s")
    NB = 8
    PB = 6
    BT = BATCH // G

    @functools.partial(
        pl.kernel,
        mesh=mesh,
        out_type=jax.ShapeDtypeStruct((HIST_LEN, D, BATCH), jnp.float32),
        compiler_params=pltpu.CompilerParams(use_tc_tiling_on_sc=True),
        scratch_types=[
            pltpu.VMEM((NB, D, G), jnp.float32),
            pltpu.SemaphoreType.DMA,
            pltpu.SemaphoreType.DMA,
        ],
    )
    def retile(in_hbm, out_hbm, buf, isem, osem):
        wid = lax.axis_index("s") * NC + lax.axis_index("c")
        base = wid * GROUPS

        def out_slice(j):
            t = lax.div(j, BT)
            bt = lax.rem(j, BT)
            return out_hbm.at[t, :, pl.ds(bt * G, G)]

        for p in range(PB):
            pltpu.async_copy(in_hbm.at[base + p], buf.at[p], isem)

        def step(g, carry):
            b = lax.rem(g, NB)

            @pl.when(g + PB >= NB)
            def _():
                pltpu.make_async_copy(
                    buf.at[lax.rem(g + PB, NB)], out_slice(base + g + PB - NB), osem
                ).wait()

            @pl.when(g + PB < GROUPS)
            def _():
                pltpu.async_copy(
                    in_hbm.at[base + g + PB], buf.at[lax.rem(g + PB, NB)], isem
                )

            pltpu.make_async_copy(in_hbm.at[base + g], buf.at[b], isem).wait()
            pltpu.async_copy(buf.at[b], out_slice(base + g), osem)
            return carry

        lax.fori_loop(0, GROUPS, step, 0)

        for g in range(GROUPS - NB + PB, GROUPS):
            pltpu.make_async_copy(
                buf.at[g % NB], out_slice(base + g), osem
            ).wait()

    return retile


_gather_t = _make_gather_t()
_retile = _make_retile()
_format_w = _make_format()


def kernel(token_ids, W):
    idx = jnp.transpose(token_ids).reshape(NW * GROUPS, G).astype(jnp.int32)
    wt = jnp.transpose(W)
    tail = W[(NUM_EMBEDDINGS // G) * G:].reshape(-1)
    z = jnp.zeros((8 * 16,), jnp.int32)
    table = _format_w(wt, tail, z).reshape(NUM_EMBEDDINGS, D)
    flat = _gather_t(idx, table)
    blocks = flat.reshape(NW * GROUPS, D, G)
    out3 = _retile(blocks)
    return jnp.transpose(out3, (2, 0, 1))


# speedup vs baseline: 2.9099x; 1.0007x over previous
import functools

import jax
import jax.numpy as jnp
from jax import lax
from jax.experimental import pallas as pl
from jax.experimental.pallas import tpu as pltpu
from jax.experimental.pallas import tpu_sc as plsc

NUM_EMBEDDINGS = 1000000
D = 32
BATCH = 4096
HIST_LEN = 200
B = BATCH * HIST_LEN

NC = 2
NS = 16
NW = NC * NS
G = 128
GROUPS = B // (NW * G)
NR = 8
NP = 6
NT = 2


def _make_gather_t():
    mesh = plsc.VectorSubcoreMesh(core_axis_name="c", subcore_axis_name="s")

    @functools.partial(
        pl.kernel,
        mesh=mesh,
        out_type=jax.ShapeDtypeStruct((B * D,), jnp.float32),
        compiler_params=pltpu.CompilerParams(
            use_tc_tiling_on_sc=False, needs_layout_passes=False
        ),
        scratch_types=[
            pltpu.VMEM((GROUPS, G), jnp.int32),
            pltpu.VMEM((NR, G, D), jnp.float32),
            pltpu.VMEM((NT, D * G), jnp.float32),
            pltpu.SemaphoreType.DMA,
            pltpu.SemaphoreType.DMA,
        ],
    )
    def gather_t(idx_hbm, table_hbm, out_hbm, idx_v, rows_v, trans_v, gsem, osem):
        wid = lax.axis_index("s") * NC + lax.axis_index("c")
        base = wid * GROUPS
        pltpu.sync_copy(idx_hbm.at[pl.ds(base, GROUPS)], idx_v)

        iota = lax.iota(jnp.int32, 16)

        for p in range(NP):
            pltpu.async_copy(table_hbm.at[idx_v.at[p]], rows_v.at[p], gsem)

        def step(g, carry):
            rb = lax.rem(g, NR)
            tb = lax.rem(g, NT)

            @pl.when(g >= NT)
            def _():
                pltpu.make_async_copy(
                    trans_v.at[tb],
                    out_hbm.at[pl.ds((base + g - NT) * G * D, G * D)],
                    osem,
                ).wait()

            @pl.when(g + NP < GROUPS)
            def _():
                pltpu.async_copy(
                    table_hbm.at[idx_v.at[g + NP]],
                    rows_v.at[lax.rem(g + NP, NR)],
                    gsem,
                )

            pltpu.make_async_copy(
                table_hbm.at[idx_v.at[g]], rows_v.at[rb], gsem
            ).wait()

            rv = rows_v.at[rb]
            tv = trans_v.at[tb]
            iz = lax.shift_right_arithmetic(idx_v[g, pl.ds(0, 16)], 31)
            iota_p = iota + iz
            subtiles = [(c, h) for c in range(8) for h in range(2)]

            def loads(c, h):
                out = []
                row = iota_p + c * 16
                for d in range(16):
                    cb = lax.bitwise_and(iota_p + d, jnp.int32(15))
                    x = plsc.load_gather(rv, [row, cb + h * 16])
                    out.append((cb, x))
                return out

            def stores(c, h, xs):
                off = iota_p + (h * 16 * G + c * 16)
                for cb, x in xs:
                    plsc.store_scatter(tv, [lax.shift_left(cb, 7) + off], x)

            prev_ch, prev_xs = subtiles[0], loads(*subtiles[0])
            for ch in subtiles[1:]:
                cur = loads(*ch)
                stores(*prev_ch, prev_xs)
                prev_ch, prev_xs = ch, cur
            stores(*prev_ch, prev_xs)

            pltpu.async_copy(
                tv, out_hbm.at[pl.ds((base + g) * G * D, G * D)], osem
            )
            return carry

        lax.fori_loop(0, GROUPS, step, 0)

        for g in (GROUPS - NT, GROUPS - 1):
            pltpu.make_async_copy(
                trans_v.at[g % NT],
                out_hbm.at[pl.ds((base + g) * G * D, G * D)],
                osem,
            ).wait()

    return gather_t


def _make_format():
    mesh = plsc.VectorSubcoreMesh(core_axis_name="c", subcore_axis_name="s")
    CHUNKS = NUM_EMBEDDINGS // G
    TAIL = NUM_EMBEDDINGS - CHUNKS * G
    NK = 8
    KP = 6
    NO = 2

    @functools.partial(
        pl.kernel,
        mesh=mesh,
        out_type=jax.ShapeDtypeStruct((NUM_EMBEDDINGS * D,), jnp.float32),
        compiler_params=pltpu.CompilerParams(
            use_tc_tiling_on_sc=True, needs_layout_passes=False
        ),
        scratch_types=[
            pltpu.VMEM((NK * D, G), jnp.float32),
            pltpu.VMEM((NO * D * G,), jnp.float32),
            pltpu.VMEM((NK * 16,), jnp.int32),
            pltpu.VMEM((TAIL * D,), jnp.float32),
            pltpu.SemaphoreType.DMA,
            pltpu.SemaphoreType.DMA,
        ],
    )
    def format_w(wt_hbm, tail_hbm, z_hbm, out_hbm, inb, outb, zbuf, tailv, isem, osem):
        wid = lax.axis_index("s") * NC + lax.axis_index("c")
        pltpu.sync_copy(z_hbm, zbuf)

        @pl.when(wid == NW - 1)
        def _():
            pltpu.sync_copy(tail_hbm, tailv)
            pltpu.sync_copy(tailv, out_hbm.at[pl.ds(CHUNKS * G * D, TAIL * D)])

        nw = jnp.where(wid < CHUNKS - (CHUNKS // NW) * NW, CHUNKS // NW + 1,
                       CHUNKS // NW).astype(jnp.int32)
        iota = lax.iota(jnp.int32, 16)

        def src(i):
            return wt_hbm.at[:, pl.ds((i * NW + wid) * G, G)]

        for p2 in range(KP):
            pltpu.async_copy(src(p2), inb.at[pl.ds(p2 * D, D)], isem)

        def step(i, carry):
            rb = lax.rem(i, NK)
            ob = lax.rem(i, NO)

            @pl.when(i >= NO)
            def _():
                pltpu.make_async_copy(
                    outb.at[pl.ds(ob * D * G, D * G)],
                    out_hbm.at[pl.ds(((i - NO) * NW + wid) * G * D, G * D)],
                    osem,
                ).wait()

            @pl.when(i + KP < nw)
            def _():
                pltpu.async_copy(src(i + KP), inb.at[pl.ds(lax.rem(i + KP, NK) * D, D)], isem)

            pltpu.make_async_copy(src(i), inb.at[pl.ds(rb * D, D)], isem).wait()

            rv = inb.at[pl.ds(rb * D, D)]
            tv = outb.at[pl.ds(ob * D * G, D * G)]
            iz = lax.shift_right_arithmetic(
                zbuf[pl.ds(lax.rem(i, NK) * 16, 16)], 31)
            iota_p = iota + iz
            subtiles = [(rg, cg) for rg in range(2) for cg in range(8)]

            def loads(rg, cg):
                out = []
                row = iota_p + rg * 16
                for k in range(16):
                    cb = lax.bitwise_and(iota_p + k, jnp.int32(15))
                    x = plsc.load_gather(rv, [row, cb + cg * 16])
                    out.append((cb, x))
                return out

            def stores(rg, cg, xs):
                off = iota_p + rg * 16
                for cb, x in xs:
                    sidx = lax.shift_left(cb + cg * 16, 5) + off
                    plsc.store_scatter(tv, [sidx], x)

            prev_ch, prev_xs = subtiles[0], loads(*subtiles[0])
            for ch in subtiles[1:]:
                cur = loads(*ch)
                stores(*prev_ch, prev_xs)
                prev_ch, prev_xs = ch, cur
            stores(*prev_ch, prev_xs)

            pltpu.async_copy(
                tv, out_hbm.at[pl.ds((i * NW + wid) * G * D, G * D)], osem
            )
            return carry

        lax.fori_loop(0, nw, step, jnp.int32(0))

        for back in (NO, 1):
            pltpu.make_async_copy(
                outb.at[pl.ds(lax.rem(nw - back, NO) * D * G, D * G)],
                out_hbm.at[pl.ds(((nw - back) * NW + wid) * G * D, G * D)],
                osem,
            ).wait()

    return format_w


def _make_retile():
    mesh = plsc.VectorSubcoreMesh(core_axis_name="c", subcore_axis_name="s")
    NB = 8
    PB = 6
    BT = BATCH // G

    @functools.partial(
        pl.kernel,
        mesh=mesh,
        out_type=jax.ShapeDtypeStruct((HIST_LEN, D, BATCH), jnp.float32),
        compiler_params=pltpu.CompilerParams(use_tc_tiling_on_sc=True),
        scratch_types=[
            pltpu.VMEM((NB, D, G), jnp.float32),
            pltpu.SemaphoreType.DMA,
            pltpu.SemaphoreType.DMA,
        ],
    )
    def retile(in_hbm, out_hbm, buf, isem, osem):
        wid = lax.axis_index("s") * NC + lax.axis_index("c")
        base = wid * GROUPS

        def out_slice(j):
            t = lax.div(j, BT)
            bt = lax.rem(j, BT)
            return out_hbm.at[t, :, pl.ds(bt * G, G)]

        for p in range(PB):
            pltpu.async_copy(in_hbm.at[base + p], buf.at[p], isem)

        def step(g, carry):
            b = lax.rem(g, NB)

            @pl.when(g + PB >= NB)
            def _():
                pltpu.make_async_copy(
                    buf.at[lax.rem(g + PB, NB)], out_slice(base + g + PB - NB), osem
                ).wait()

            @pl.when(g + PB < GROUPS)
            def _():
                pltpu.async_copy(
                    in_hbm.at[base + g + PB], buf.at[lax.rem(g + PB, NB)], isem
                )

            pltpu.make_async_copy(in_hbm.at[base + g], buf.at[b], isem).wait()
            pltpu.async_copy(buf.at[b], out_slice(base + g), osem)
            return carry

        lax.fori_loop(0, GROUPS, step, 0)

        for g in range(GROUPS - NB + PB, GROUPS):
            pltpu.make_async_copy(
                buf.at[g % NB], out_slice(base + g), osem
            ).wait()

    return retile


_gather_t = _make_gather_t()
_retile = _make_retile()
_format_w = _make_format()


def kernel(token_ids, W):
    idx = jnp.transpose(token_ids).reshape(NW * GROUPS, G).astype(jnp.int32)
    wt = jnp.transpose(W)
    tail = W[(NUM_EMBEDDINGS // G) * G:].reshape(-1)
    z = jnp.zeros((8 * 16,), jnp.int32)
    table = _format_w(wt, tail, z).reshape(NUM_EMBEDDINGS, D)
    flat = _gather_t(idx, table)
    blocks = flat.reshape(NW * GROUPS, D, G)
    out3 = _retile(blocks)
    return jnp.transpose(out3, (2, 0, 1))
